# baseline (speedup 1.0000x reference)
import dataclasses
import functools

import jax
import jax.numpy as jnp
from jax import lax
from jax.experimental import pallas as pl
from jax.experimental.pallas import tpu as pltpu
from jax.experimental.pallas import tpu_sc as plsc

B, N, K = 4, 4096, 16
RBLK = 256
BLK = 64
NB = N // BLK
NW = 32
CH = 8
STEPS = N // (NW * CH)
L = 16
FMAX = 3.4028235e38


def _dist_body(pts_ref, ptsT_ref, d_ref, bm_ref):
    xi = pts_ref[0, :, 0:1]
    yi = pts_ref[0, :, 1:2]
    zi = pts_ref[0, :, 2:3]
    xj = ptsT_ref[0, 0:1, :]
    yj = ptsT_ref[0, 1:2, :]
    zj = ptsT_ref[0, 2:3, :]
    bf = jnp.bfloat16
    f32 = jnp.float32
    xbi, ybi, zbi = (c.astype(bf).astype(f32) for c in (xi, yi, zi))
    xbj, ybj, zbj = (c.astype(bf).astype(f32) for c in (xj, yj, zj))
    dot = xbi * xbj + ybi * ybj + zbi * zbj
    sqi = xi * xi + yi * yi + zi * zi
    sqj = xj * xj + yj * yj + zj * zj
    d = (sqi + sqj) - 2.0 * dot
    d_ref[0, :, :] = d
    bm_ref[0, :, :] = jnp.min(d.reshape(RBLK, NB, BLK), axis=-1)


def _pairwise_sq_dists(points):
    ptsT = jnp.transpose(points, (0, 2, 1))
    return pl.pallas_call(
        _dist_body,
        grid=(1, N // RBLK),
        in_specs=[
            pl.BlockSpec((1, RBLK, 3), lambda b, i: (b, i, 0)),
            pl.BlockSpec((1, 3, N), lambda b, i: (b, 0, 0)),
        ],
        out_specs=[
            pl.BlockSpec((1, RBLK, N), lambda b, i: (b, i, 0)),
            pl.BlockSpec((1, RBLK, NB), lambda b, i: (b, i, 0)),
        ],
        out_shape=[
            jax.ShapeDtypeStruct((1, N, N), jnp.float32),
            jax.ShapeDtypeStruct((1, N, NB), jnp.float32),
        ],
    )(points, ptsT)


def _topk_row(d_vmem, bm_vmem, cand_j, out_vmem, r, consts):
    iota, iotas, f15, inf_v, zero_v = consts
    rv = jnp.full((L,), r, jnp.int32)

    def merge16(a, b):
        ka, va = a
        kb, vb = b
        rkb = lax.rev(kb, (0,))
        rvb = lax.rev(vb, (0,))
        take = ka <= rkb
        return jnp.where(take, ka, rkb), jnp.where(take, va, rvb)

    sorted_bms = [
        plsc.sort_key_val(bm_vmem[r, pl.ds(h * L, L)], iota + h * L)
        for h in range(NB // L)
    ]
    while len(sorted_bms) > 1:
        nxt = []
        for i in range(0, len(sorted_bms), 2):
            mkv = merge16(sorted_bms[i], sorted_bms[i + 1])
            if len(sorted_bms) > 2:
                mkv = plsc.sort_key_val(*mkv)
            nxt.append(mkv)
        sorted_bms = nxt
    mk, mv = sorted_bms[0]
    tau_v = jnp.take(plsc.cummax(mk), f15)

    mv128 = mv * BLK
    prefix = jnp.full((L,), -1, jnp.int32)
    for s in range(L):
        base_v = jnp.take(mv128, jnp.full((L,), s, jnp.int32))
        for p in range(BLK // L):
            idxv = base_v + iotas[p]
            dv = plsc.load_gather(d_vmem, [rv, idxv])
            le = dv <= tau_v
            cum = plsc.cumsum(le.astype(jnp.int32))
            pos = prefix + cum
            plsc.store_scatter(cand_j, [pos], idxv, mask=le)
            prefix = prefix + jnp.take(cum, f15)

    w = jnp.max(prefix) + 1
    w_v = prefix
    nchunks = (w + (L - 1)) // L

    def p3_body(t, carry):
        bd, bj = carry
        valid = (iota + t * L) <= w_v
        cj = jnp.where(valid, cand_j[pl.ds(t * L, L)], 0)
        cd = plsc.load_gather(d_vmem, [rv, cj])
        cd = jnp.where(valid, cd, FMAX)
        scd, scj = plsc.sort_key_val(cd, cj)
        rb = lax.rev(bd, (0,))
        rbj = lax.rev(bj, (0,))
        tk = scd <= rb
        nd = jnp.where(tk, scd, rb)
        nj = jnp.where(tk, scj, rbj)
        bd, bj = plsc.sort_key_val(nd, nj)
        return bd, bj

    bd, bj = lax.fori_loop(0, nchunks, p3_body, (inf_v, zero_v))
    out_vmem[r, :] = bj


def _sc_topk(d, bm):
    mesh = plsc.VectorSubcoreMesh(core_axis_name="core",
                                  subcore_axis_name="subcore")
    cp = pltpu.CompilerParams()
    if "needs_layout_passes" in pltpu.CompilerParams.__dataclass_fields__:
        cp = dataclasses.replace(cp, needs_layout_passes=False)

    @functools.partial(
        pl.kernel,
        compiler_params=cp,
        out_type=jax.ShapeDtypeStruct((N, K), jnp.int32),
        mesh=mesh,
        scratch_types=[
            pltpu.VMEM((N,), jnp.int32),
            pltpu.VMEM((N,), jnp.int32),
        ],
    )
    def sc_kernel(d_hbm, bm_hbm, out_hbm, cand_j, cand_j2):
        def body(d_vmem, bm_vmem, out_vmem):
            iota = lax.iota(jnp.int32, L)
            consts = (
                iota,
                [iota + p * L for p in range(BLK // L)],
                jnp.full((L,), L - 1, jnp.int32),
                jnp.full((L,), FMAX, jnp.float32),
                jnp.zeros((L,), jnp.int32),
            )

            @pl.loop(0, CH, step=2)
            def row_body(r):
                _topk_row(d_vmem, bm_vmem, cand_j, out_vmem, r, consts)
                _topk_row(d_vmem, bm_vmem, cand_j2, out_vmem, r + 1, consts)

        pltpu.emit_pipeline(
            body,
            grid=(NW, STEPS),
            in_specs=[
                pl.BlockSpec((CH, N), lambda i, j: (i * STEPS + j, 0)),
                pl.BlockSpec((CH, NB), lambda i, j: (i * STEPS + j, 0)),
            ],
            out_specs=[pl.BlockSpec((CH, K), lambda i, j: (i * STEPS + j, 0))],
            core_axis_name=("core", "subcore"),
            dimension_semantics=(pltpu.PARALLEL, pltpu.ARBITRARY),
        )(d_hbm, bm_hbm, out_hbm)

    return sc_kernel(d, bm)


def kernel(points):
    idxs = []
    for b in range(B):
        d, bm = _pairwise_sq_dists(points[b:b + 1])
        idxs.append(_sc_topk(d.reshape(N, N), bm.reshape(N, NB)))
    return jnp.stack(idxs, axis=0)


# speedup vs baseline: 1.2620x; 1.2620x over previous
import dataclasses
import functools

import jax
import jax.numpy as jnp
from jax import lax
from jax.experimental import pallas as pl
from jax.experimental.pallas import tpu as pltpu
from jax.experimental.pallas import tpu_sc as plsc

B, N, K = 4, 4096, 16
RBLK = 256
BLK = 32
NB = N // BLK
NW = 32
CH = 8
STEPS = N // (NW * CH)
L = 16
FMAX = 3.4028235e38


def _dist_body(pts_ref, ptsT_ref, d_ref, bm_ref):
    xi = pts_ref[0, :, 0:1]
    yi = pts_ref[0, :, 1:2]
    zi = pts_ref[0, :, 2:3]
    xj = ptsT_ref[0, 0:1, :]
    yj = ptsT_ref[0, 1:2, :]
    zj = ptsT_ref[0, 2:3, :]
    bf = jnp.bfloat16
    f32 = jnp.float32
    xbi, ybi, zbi = (c.astype(bf).astype(f32) for c in (xi, yi, zi))
    xbj, ybj, zbj = (c.astype(bf).astype(f32) for c in (xj, yj, zj))
    dot = xbi * xbj + ybi * ybj + zbi * zbj
    sqi = xi * xi + yi * yi + zi * zi
    sqj = xj * xj + yj * yj + zj * zj
    d = (sqi + sqj) - 2.0 * dot
    d_ref[0, :, :] = d
    bm_ref[0, :, :] = jnp.min(d.reshape(RBLK, NB, BLK), axis=-1)


def _pairwise_sq_dists(points):
    ptsT = jnp.transpose(points, (0, 2, 1))
    return pl.pallas_call(
        _dist_body,
        grid=(1, N // RBLK),
        in_specs=[
            pl.BlockSpec((1, RBLK, 3), lambda b, i: (b, i, 0)),
            pl.BlockSpec((1, 3, N), lambda b, i: (b, 0, 0)),
        ],
        out_specs=[
            pl.BlockSpec((1, RBLK, N), lambda b, i: (b, i, 0)),
            pl.BlockSpec((1, RBLK, NB), lambda b, i: (b, i, 0)),
        ],
        out_shape=[
            jax.ShapeDtypeStruct((1, N, N), jnp.float32),
            jax.ShapeDtypeStruct((1, N, NB), jnp.float32),
        ],
    )(points, ptsT)


def _topk_row(d_vmem, bm_vmem, cand_j, out_vmem, r, consts):
    iota, iotas, f15, inf_v, zero_v = consts
    rv = jnp.full((L,), r, jnp.int32)

    def merge16(a, b):
        ka, va = a
        kb, vb = b
        rkb = lax.rev(kb, (0,))
        rvb = lax.rev(vb, (0,))
        take = ka <= rkb
        return jnp.where(take, ka, rkb), jnp.where(take, va, rvb)

    sorted_bms = [
        plsc.sort_key_val(bm_vmem[r, pl.ds(h * L, L)], iota + h * L)
        for h in range(NB // L)
    ]
    while len(sorted_bms) > 1:
        nxt = []
        for i in range(0, len(sorted_bms), 2):
            mkv = merge16(sorted_bms[i], sorted_bms[i + 1])
            if len(sorted_bms) > 2:
                mkv = plsc.sort_key_val(*mkv)
            nxt.append(mkv)
        sorted_bms = nxt
    mk, mv = sorted_bms[0]
    tau_v = jnp.take(plsc.cummax(mk), f15)

    mv128 = mv * BLK
    prefix = jnp.full((L,), -1, jnp.int32)
    for s in range(L):
        base_v = jnp.take(mv128, jnp.full((L,), s, jnp.int32))
        for p in range(BLK // L):
            idxv = base_v + iotas[p]
            dv = plsc.load_gather(d_vmem, [rv, idxv])
            le = dv <= tau_v
            cum = plsc.cumsum(le.astype(jnp.int32))
            pos = prefix + cum
            plsc.store_scatter(cand_j, [pos], idxv, mask=le)
            prefix = prefix + jnp.take(cum, f15)

    w = jnp.max(prefix) + 1
    w_v = prefix
    nchunks = (w + (L - 1)) // L

    def p3_body(t, carry):
        bd, bj = carry
        valid = (iota + t * L) <= w_v
        cj = jnp.where(valid, cand_j[pl.ds(t * L, L)], 0)
        cd = plsc.load_gather(d_vmem, [rv, cj])
        cd = jnp.where(valid, cd, FMAX)
        scd, scj = plsc.sort_key_val(cd, cj)
        rb = lax.rev(bd, (0,))
        rbj = lax.rev(bj, (0,))
        tk = scd <= rb
        nd = jnp.where(tk, scd, rb)
        nj = jnp.where(tk, scj, rbj)
        bd, bj = plsc.sort_key_val(nd, nj)
        return bd, bj

    bd, bj = lax.fori_loop(0, nchunks, p3_body, (inf_v, zero_v))
    out_vmem[r, :] = bj


def _sc_topk(d, bm):
    mesh = plsc.VectorSubcoreMesh(core_axis_name="core",
                                  subcore_axis_name="subcore")
    cp = pltpu.CompilerParams()
    if "needs_layout_passes" in pltpu.CompilerParams.__dataclass_fields__:
        cp = dataclasses.replace(cp, needs_layout_passes=False)

    @functools.partial(
        pl.kernel,
        compiler_params=cp,
        out_type=jax.ShapeDtypeStruct((N, K), jnp.int32),
        mesh=mesh,
        scratch_types=[
            pltpu.VMEM((N,), jnp.int32),
        ],
    )
    def sc_kernel(d_hbm, bm_hbm, out_hbm, cand_j):
        def body(d_vmem, bm_vmem, out_vmem):
            iota = lax.iota(jnp.int32, L)
            consts = (
                iota,
                [iota + p * L for p in range(BLK // L)],
                jnp.full((L,), L - 1, jnp.int32),
                jnp.full((L,), FMAX, jnp.float32),
                jnp.zeros((L,), jnp.int32),
            )

            @pl.loop(0, CH)
            def row_body(r):
                _topk_row(d_vmem, bm_vmem, cand_j, out_vmem, r, consts)

        pltpu.emit_pipeline(
            body,
            grid=(NW, STEPS),
            in_specs=[
                pl.BlockSpec((CH, N), lambda i, j: (i * STEPS + j, 0)),
                pl.BlockSpec((CH, NB), lambda i, j: (i * STEPS + j, 0)),
            ],
            out_specs=[pl.BlockSpec((CH, K), lambda i, j: (i * STEPS + j, 0))],
            core_axis_name=("core", "subcore"),
            dimension_semantics=(pltpu.PARALLEL, pltpu.ARBITRARY),
        )(d_hbm, bm_hbm, out_hbm)

    return sc_kernel(d, bm)


def kernel(points):
    idxs = []
    for b in range(B):
        d, bm = _pairwise_sq_dists(points[b:b + 1])
        idxs.append(_sc_topk(d.reshape(N, N), bm.reshape(N, NB)))
    return jnp.stack(idxs, axis=0)


# speedup vs baseline: 1.4445x; 1.1446x over previous
import dataclasses
import functools

import jax
import jax.numpy as jnp
from jax import lax
from jax.experimental import pallas as pl
from jax.experimental.pallas import tpu as pltpu
from jax.experimental.pallas import tpu_sc as plsc

B, N, K = 4, 4096, 16
RBLK = 256
BLK = 64
NB = N // BLK
NW = 32
CH = 8
STEPS = N // (NW * CH)
L = 16
FMAX = 3.4028235e38


def _dist_body(pts_ref, ptsT_ref, d_ref, bm_ref):
    xi = pts_ref[0, :, 0:1]
    yi = pts_ref[0, :, 1:2]
    zi = pts_ref[0, :, 2:3]
    xj = ptsT_ref[0, 0:1, :]
    yj = ptsT_ref[0, 1:2, :]
    zj = ptsT_ref[0, 2:3, :]
    bf = jnp.bfloat16
    f32 = jnp.float32
    xbi, ybi, zbi = (c.astype(bf).astype(f32) for c in (xi, yi, zi))
    xbj, ybj, zbj = (c.astype(bf).astype(f32) for c in (xj, yj, zj))
    dot = xbi * xbj + ybi * ybj + zbi * zbj
    sqi = xi * xi + yi * yi + zi * zi
    sqj = xj * xj + yj * yj + zj * zj
    d = (sqi + sqj) - 2.0 * dot
    d_ref[0, :, :] = d
    bm_ref[0, :, :] = jnp.min(d.reshape(RBLK, NB, BLK), axis=-1)


def _pairwise_sq_dists(points):
    ptsT = jnp.transpose(points, (0, 2, 1))
    return pl.pallas_call(
        _dist_body,
        grid=(1, N // RBLK),
        in_specs=[
            pl.BlockSpec((1, RBLK, 3), lambda b, i: (b, i, 0)),
            pl.BlockSpec((1, 3, N), lambda b, i: (b, 0, 0)),
        ],
        out_specs=[
            pl.BlockSpec((1, RBLK, N), lambda b, i: (b, i, 0)),
            pl.BlockSpec((1, RBLK, NB), lambda b, i: (b, i, 0)),
        ],
        out_shape=[
            jax.ShapeDtypeStruct((1, N, N), jnp.float32),
            jax.ShapeDtypeStruct((1, N, NB), jnp.float32),
        ],
    )(points, ptsT)


def _topk_row(d_vmem, bm_vmem, cand_j, out_vmem, r, consts):
    iota, iotas, f15, inf_v, zero_v = consts
    rv = jnp.full((L,), r, jnp.int32)

    def merge16(a, b):
        ka, va = a
        kb, vb = b
        rkb = lax.rev(kb, (0,))
        rvb = lax.rev(vb, (0,))
        take = ka <= rkb
        return jnp.where(take, ka, rkb), jnp.where(take, va, rvb)

    sorted_bms = [
        plsc.sort_key_val(bm_vmem[r, pl.ds(h * L, L)], iota + h * L)
        for h in range(NB // L)
    ]
    while len(sorted_bms) > 1:
        nxt = []
        for i in range(0, len(sorted_bms), 2):
            mkv = merge16(sorted_bms[i], sorted_bms[i + 1])
            if len(sorted_bms) > 2:
                mkv = plsc.sort_key_val(*mkv)
            nxt.append(mkv)
        sorted_bms = nxt
    mk, mv = sorted_bms[0]
    tau_v = jnp.take(plsc.cummax(mk), f15)

    mv128 = mv * BLK
    prefix = jnp.full((L,), -1, jnp.int32)
    for s in range(L):
        base_v = jnp.take(mv128, jnp.full((L,), s, jnp.int32))
        for p in range(BLK // L):
            idxv = base_v + iotas[p]
            dv = plsc.load_gather(d_vmem, [rv, idxv])
            le = dv <= tau_v
            cum = plsc.cumsum(le.astype(jnp.int32))
            pos = prefix + cum
            plsc.store_scatter(cand_j, [pos], idxv, mask=le)
            prefix = prefix + jnp.take(cum, f15)

    w = jnp.max(prefix) + 1
    w_v = prefix
    nchunks = (w + (L - 1)) // L

    def p3_body(t, carry):
        bd, bj = carry
        valid = (iota + t * L) <= w_v
        cj = jnp.where(valid, cand_j[pl.ds(t * L, L)], 0)
        cd = plsc.load_gather(d_vmem, [rv, cj])
        cd = jnp.where(valid, cd, FMAX)
        scd, scj = plsc.sort_key_val(cd, cj)
        rb = lax.rev(bd, (0,))
        rbj = lax.rev(bj, (0,))
        tk = scd <= rb
        nd = jnp.where(tk, scd, rb)
        nj = jnp.where(tk, scj, rbj)
        bd, bj = plsc.sort_key_val(nd, nj)
        return bd, bj

    bd, bj = lax.fori_loop(0, nchunks, p3_body, (inf_v, zero_v))
    out_vmem[r, :] = bj


def _sc_topk(d, bm):
    mesh = plsc.VectorSubcoreMesh(core_axis_name="core",
                                  subcore_axis_name="subcore")
    cp = pltpu.CompilerParams()
    if "needs_layout_passes" in pltpu.CompilerParams.__dataclass_fields__:
        cp = dataclasses.replace(cp, needs_layout_passes=False)

    @functools.partial(
        pl.kernel,
        compiler_params=cp,
        out_type=jax.ShapeDtypeStruct((N, K), jnp.int32),
        mesh=mesh,
        scratch_types=[
            pltpu.VMEM((N,), jnp.int32),
        ],
    )
    def sc_kernel(d_hbm, bm_hbm, out_hbm, cand_j):
        def body(d_vmem, bm_vmem, out_vmem):
            iota = lax.iota(jnp.int32, L)
            consts = (
                iota,
                [iota + p * L for p in range(BLK // L)],
                jnp.full((L,), L - 1, jnp.int32),
                jnp.full((L,), FMAX, jnp.float32),
                jnp.zeros((L,), jnp.int32),
            )

            @pl.loop(0, CH)
            def row_body(r):
                _topk_row(d_vmem, bm_vmem, cand_j, out_vmem, r, consts)

        pltpu.emit_pipeline(
            body,
            grid=(NW, STEPS),
            in_specs=[
                pl.BlockSpec((CH, N), lambda i, j: (i * STEPS + j, 0)),
                pl.BlockSpec((CH, NB), lambda i, j: (i * STEPS + j, 0)),
            ],
            out_specs=[pl.BlockSpec((CH, K), lambda i, j: (i * STEPS + j, 0))],
            core_axis_name=("core", "subcore"),
            dimension_semantics=(pltpu.PARALLEL, pltpu.ARBITRARY),
        )(d_hbm, bm_hbm, out_hbm)

    return sc_kernel(d, bm)


def kernel(points):
    idxs = []
    for b in range(B):
        d, bm = _pairwise_sq_dists(points[b:b + 1])
        idxs.append(_sc_topk(d.reshape(N, N), bm.reshape(N, NB)))
    return jnp.stack(idxs, axis=0)


# speedup vs baseline: 1.4527x; 1.0057x over previous
import dataclasses
import functools

import jax
import jax.numpy as jnp
from jax import lax
from jax.experimental import pallas as pl
from jax.experimental.pallas import tpu as pltpu
from jax.experimental.pallas import tpu_sc as plsc

B, N, K = 4, 4096, 16
RBLK = 256
BLK = 64
NB = N // BLK
NW = 32
CH = 4
STEPS = N // (NW * CH)
L = 16
FMAX = 3.4028235e38


def _dist_body(pts_ref, ptsT_ref, d_ref, bm_ref):
    xi = pts_ref[0, :, 0:1]
    yi = pts_ref[0, :, 1:2]
    zi = pts_ref[0, :, 2:3]
    xj = ptsT_ref[0, 0:1, :]
    yj = ptsT_ref[0, 1:2, :]
    zj = ptsT_ref[0, 2:3, :]
    bf = jnp.bfloat16
    f32 = jnp.float32
    xbi, ybi, zbi = (c.astype(bf).astype(f32) for c in (xi, yi, zi))
    xbj, ybj, zbj = (c.astype(bf).astype(f32) for c in (xj, yj, zj))
    dot = xbi * xbj + ybi * ybj + zbi * zbj
    sqi = xi * xi + yi * yi + zi * zi
    sqj = xj * xj + yj * yj + zj * zj
    d = (sqi + sqj) - 2.0 * dot
    d_ref[0, :, :] = d
    bm_ref[0, :, :] = jnp.min(d.reshape(RBLK, NB, BLK), axis=-1)


def _pairwise_sq_dists(points):
    ptsT = jnp.transpose(points, (0, 2, 1))
    return pl.pallas_call(
        _dist_body,
        grid=(1, N // RBLK),
        in_specs=[
            pl.BlockSpec((1, RBLK, 3), lambda b, i: (b, i, 0)),
            pl.BlockSpec((1, 3, N), lambda b, i: (b, 0, 0)),
        ],
        out_specs=[
            pl.BlockSpec((1, RBLK, N), lambda b, i: (b, i, 0)),
            pl.BlockSpec((1, RBLK, NB), lambda b, i: (b, i, 0)),
        ],
        out_shape=[
            jax.ShapeDtypeStruct((1, N, N), jnp.float32),
            jax.ShapeDtypeStruct((1, N, NB), jnp.float32),
        ],
    )(points, ptsT)


def _topk_row(d_vmem, bm_vmem, cand_j, out_vmem, r, consts):
    iota, iotas, f15, inf_v, zero_v = consts
    rv = jnp.full((L,), r, jnp.int32)

    def merge16(a, b):
        ka, va = a
        kb, vb = b
        rkb = lax.rev(kb, (0,))
        rvb = lax.rev(vb, (0,))
        take = ka <= rkb
        return jnp.where(take, ka, rkb), jnp.where(take, va, rvb)

    sorted_bms = [
        plsc.sort_key_val(bm_vmem[r, pl.ds(h * L, L)], iota + h * L)
        for h in range(NB // L)
    ]
    while len(sorted_bms) > 1:
        nxt = []
        for i in range(0, len(sorted_bms), 2):
            mkv = merge16(sorted_bms[i], sorted_bms[i + 1])
            if len(sorted_bms) > 2:
                mkv = plsc.sort_key_val(*mkv)
            nxt.append(mkv)
        sorted_bms = nxt
    mk, mv = sorted_bms[0]
    tau_v = jnp.take(plsc.cummax(mk), f15)

    mv128 = mv * BLK
    prefix = jnp.full((L,), -1, jnp.int32)
    for s in range(L):
        base_v = jnp.take(mv128, jnp.full((L,), s, jnp.int32))
        for p in range(BLK // L):
            idxv = base_v + iotas[p]
            dv = plsc.load_gather(d_vmem, [rv, idxv])
            le = dv <= tau_v
            cum = plsc.cumsum(le.astype(jnp.int32))
            pos = prefix + cum
            plsc.store_scatter(cand_j, [pos], idxv, mask=le)
            prefix = prefix + jnp.take(cum, f15)

    w = jnp.max(prefix) + 1
    w_v = prefix
    nchunks = (w + (L - 1)) // L

    def p3_body(t, carry):
        bd, bj = carry
        valid = (iota + t * L) <= w_v
        cj = jnp.where(valid, cand_j[pl.ds(t * L, L)], 0)
        cd = plsc.load_gather(d_vmem, [rv, cj])
        cd = jnp.where(valid, cd, FMAX)
        scd, scj = plsc.sort_key_val(cd, cj)
        rb = lax.rev(bd, (0,))
        rbj = lax.rev(bj, (0,))
        tk = scd <= rb
        nd = jnp.where(tk, scd, rb)
        nj = jnp.where(tk, scj, rbj)
        bd, bj = plsc.sort_key_val(nd, nj)
        return bd, bj

    bd, bj = lax.fori_loop(0, nchunks, p3_body, (inf_v, zero_v))
    out_vmem[r, :] = bj


def _sc_topk(d, bm):
    mesh = plsc.VectorSubcoreMesh(core_axis_name="core",
                                  subcore_axis_name="subcore")
    cp = pltpu.CompilerParams()
    if "needs_layout_passes" in pltpu.CompilerParams.__dataclass_fields__:
        cp = dataclasses.replace(cp, needs_layout_passes=False)

    @functools.partial(
        pl.kernel,
        compiler_params=cp,
        out_type=jax.ShapeDtypeStruct((N, K), jnp.int32),
        mesh=mesh,
        scratch_types=[
            pltpu.VMEM((N,), jnp.int32),
        ],
    )
    def sc_kernel(d_hbm, bm_hbm, out_hbm, cand_j):
        def body(d_vmem, bm_vmem, out_vmem):
            iota = lax.iota(jnp.int32, L)
            consts = (
                iota,
                [iota + p * L for p in range(BLK // L)],
                jnp.full((L,), L - 1, jnp.int32),
                jnp.full((L,), FMAX, jnp.float32),
                jnp.zeros((L,), jnp.int32),
            )

            @pl.loop(0, CH)
            def row_body(r):
                _topk_row(d_vmem, bm_vmem, cand_j, out_vmem, r, consts)

        pltpu.emit_pipeline(
            body,
            grid=(NW, STEPS),
            in_specs=[
                pl.BlockSpec((CH, N), lambda i, j: (i * STEPS + j, 0)),
                pl.BlockSpec((CH, NB), lambda i, j: (i * STEPS + j, 0)),
            ],
            out_specs=[pl.BlockSpec((CH, K), lambda i, j: (i * STEPS + j, 0))],
            core_axis_name=("core", "subcore"),
            dimension_semantics=(pltpu.PARALLEL, pltpu.ARBITRARY),
        )(d_hbm, bm_hbm, out_hbm)

    return sc_kernel(d, bm)


def kernel(points):
    idxs = []
    for b in range(B):
        d, bm = _pairwise_sq_dists(points[b:b + 1])
        idxs.append(_sc_topk(d.reshape(N, N), bm.reshape(N, NB)))
    return jnp.stack(idxs, axis=0)
